# baseline (speedup 1.0000x reference)
import jax
import jax.numpy as jnp
from jax import lax
from jax.experimental import pallas as pl
from jax.experimental.pallas import tpu as pltpu
from jax.experimental.pallas import tpu_sc as plsc

_BATCH = 64
_SEQ_LEN = 262144
_MAX_DROP_LENGTH = 2048
_WIN = _MAX_DROP_LENGTH + 16
_NW = 32
_ROWS_PER_W = _BATCH // _NW


def _drop_bounds(batch, seq_len):
    rkey = jax.random.key(42)
    k_start, k_len = jax.random.split(rkey)
    drop_start = jax.random.randint(k_start, (batch,), 0, seq_len // 2)
    drop_len = jax.random.randint(k_len, (batch,), 0, _MAX_DROP_LENGTH)
    drop_end = jnp.minimum(drop_start + drop_len, seq_len)
    return drop_start.astype(jnp.int32), drop_end.astype(jnp.int32)


def _scalar_at(vmem_ref, i):
    base = (i // 16) * 16
    v = vmem_ref[pl.ds(base, 16)]
    lane = lax.broadcasted_iota(jnp.int32, (16,), 0)
    return jnp.max(jnp.where(lane == i % 16, v, 0), axis=0)


def _sc_fix_body(sa_hbm, s_hbm, e_hbm, buf_hbm,
                 sa_v, s_v, e_v, wbuf, sem_in, sem_out):
    wid = lax.axis_index("s") * 2 + lax.axis_index("c")
    row0 = wid * _ROWS_PER_W

    pltpu.sync_copy(sa_hbm, sa_v)
    pltpu.sync_copy(s_hbm, s_v)
    pltpu.sync_copy(e_hbm, e_v)

    sas = [pl.multiple_of(_scalar_at(sa_v, row0 + k), 16)
           for k in range(_ROWS_PER_W)]
    for k in range(_ROWS_PER_W):
        pltpu.async_copy(
            buf_hbm.at[pl.ds(sas[k], _WIN)], wbuf.at[k], sem_in[k])

    for k in range(_ROWS_PER_W):
        r = row0 + k
        s = _scalar_at(s_v, r)
        e = _scalar_at(e_v, r)
        pltpu.make_async_copy(
            buf_hbm.at[pl.ds(sas[k], _WIN)], wbuf.at[k], sem_in[k]).wait()

        def granule(g, _, k=k, s=s, e=e, sa=sas[k]):
            off = pl.multiple_of(g * 16 - sa, 16)
            col = g * 16 + lax.broadcasted_iota(jnp.int32, (16,), 0)
            val = wbuf[k, pl.ds(off, 16)]
            drop = (col >= s) & (col < e)
            wbuf[k, pl.ds(off, 16)] = jnp.where(drop, 0.0, val)
            return 0

        lax.fori_loop(s // 16, (e + 15) // 16, granule, 0)
        pltpu.async_copy(
            wbuf.at[k], buf_hbm.at[pl.ds(sas[k], _WIN)], sem_out[k])

    for k in range(_ROWS_PER_W):
        pltpu.make_async_copy(
            wbuf.at[k], buf_hbm.at[pl.ds(sas[k], _WIN)], sem_out[k]).wait()


def kernel(waveform):
    batch, seq_len = waveform.shape
    s, e = _drop_bounds(batch, seq_len)
    row_base = jnp.arange(batch, dtype=jnp.int32) * seq_len
    sa_flat = row_base + (s // 16) * 16
    s_flat = row_base + s
    e_flat = row_base + e

    mesh = plsc.VectorSubcoreMesh(core_axis_name="c", subcore_axis_name="s")
    run = pl.kernel(
        _sc_fix_body,
        mesh=mesh,
        compiler_params=pltpu.CompilerParams(use_tc_tiling_on_sc=False,
                                             needs_layout_passes=False),
        scratch_types=[
            pltpu.VMEM((_BATCH,), jnp.int32),
            pltpu.VMEM((_BATCH,), jnp.int32),
            pltpu.VMEM((_BATCH,), jnp.int32),
            pltpu.VMEM((_ROWS_PER_W, _WIN), jnp.float32),
            [pltpu.SemaphoreType.DMA] * _ROWS_PER_W,
            [pltpu.SemaphoreType.DMA] * _ROWS_PER_W,
        ],
    )
    buf = jax.new_ref(waveform.reshape(-1))
    run(sa_flat, s_flat, e_flat, buf)
    return jax.freeze(buf).reshape(batch, seq_len)


# speedup vs baseline: 1.0816x; 1.0816x over previous
import jax
import jax.numpy as jnp
from jax import lax
from jax.experimental import pallas as pl
from jax.experimental.pallas import tpu as pltpu

_BATCH = 64
_SEQ_LEN = 262144
_MAX_DROP_LENGTH = 2048
_BLK = 2048


def _drop_bounds(batch, seq_len):
    rkey = jax.random.key(42)
    k_start, k_len = jax.random.split(rkey)
    drop_start = jax.random.randint(k_start, (batch,), 0, seq_len // 2)
    drop_len = jax.random.randint(k_len, (batch,), 0, _MAX_DROP_LENGTH)
    drop_end = jnp.minimum(drop_start + drop_len, seq_len)
    return drop_start.astype(jnp.int32), drop_end.astype(jnp.int32)


def _fix_kernel(s_ref, e_ref, s2d_ref, e2d_ref, x_ref, o_ref):
    g = pl.program_id(0)
    t = pl.program_id(1)
    blk = s_ref[g * 8 + t // 2] // _BLK + t % 2
    col = blk * _BLK + lax.broadcasted_iota(jnp.int32, (8, _BLK), 1)
    s_col = s2d_ref[:, 0:1]
    e_col = e2d_ref[:, 0:1]
    mask = (col >= s_col) & (col < e_col)
    o_ref[...] = jnp.where(mask, jnp.zeros((), x_ref.dtype), x_ref[...])


def kernel(waveform):
    batch, seq_len = waveform.shape
    s, e = _drop_bounds(batch, seq_len)
    s2d = jnp.broadcast_to(s[:, None], (batch, 128))
    e2d = jnp.broadcast_to(e[:, None], (batch, 128))

    cp = jax.freeze(jax.new_ref(waveform))

    def _data_idx(g, t, s_ref, e_ref):
        return (g, s_ref[g * 8 + t // 2] // _BLK + t % 2)

    fix = pl.pallas_call(
        _fix_kernel,
        out_shape=jax.ShapeDtypeStruct((batch, seq_len), waveform.dtype),
        grid_spec=pltpu.PrefetchScalarGridSpec(
            num_scalar_prefetch=2,
            grid=(batch // 8, 16),
            in_specs=[
                pl.BlockSpec((8, 128), lambda g, t, s_ref, e_ref: (g, 0)),
                pl.BlockSpec((8, 128), lambda g, t, s_ref, e_ref: (g, 0)),
                pl.BlockSpec((8, _BLK), _data_idx),
            ],
            out_specs=pl.BlockSpec((8, _BLK), _data_idx),
        ),
        input_output_aliases={4: 0},
    )
    return fix(s, e, s2d, e2d, cp)


# speedup vs baseline: 1.0869x; 1.0048x over previous
import jax
import jax.numpy as jnp
from jax import lax
from jax.experimental import pallas as pl
from jax.experimental.pallas import tpu as pltpu

_BATCH = 64
_SEQ_LEN = 262144
_MAX_DROP_LENGTH = 2048
_BLK = 2048


def _drop_bounds(batch, seq_len):
    rkey = jax.random.key(42)
    k_start, k_len = jax.random.split(rkey)
    drop_start = jax.random.randint(k_start, (batch,), 0, seq_len // 2)
    drop_len = jax.random.randint(k_len, (batch,), 0, _MAX_DROP_LENGTH)
    drop_end = jnp.minimum(drop_start + drop_len, seq_len)
    return drop_start.astype(jnp.int32), drop_end.astype(jnp.int32)


def _fix_kernel(s_ref, e_ref, s2d_ref, e2d_ref, x_ref, cp_ref, o_ref):
    del cp_ref
    g = pl.program_id(0)
    t = pl.program_id(1)
    blk = s_ref[g * 8 + t // 2] // _BLK + t % 2
    col = blk * _BLK + lax.broadcasted_iota(jnp.int32, (8, _BLK), 1)
    s_col = s2d_ref[:, 0:1]
    e_col = e2d_ref[:, 0:1]
    mask = (col >= s_col) & (col < e_col)
    o_ref[...] = jnp.where(mask, jnp.zeros((), x_ref.dtype), x_ref[...])


def kernel(waveform):
    batch, seq_len = waveform.shape
    s, e = _drop_bounds(batch, seq_len)
    s2d = jnp.broadcast_to(s[:, None], (batch, 128))
    e2d = jnp.broadcast_to(e[:, None], (batch, 128))

    cp = jax.freeze(jax.new_ref(waveform))

    def _data_idx(g, t, s_ref, e_ref):
        return (g, s_ref[g * 8 + t // 2] // _BLK + t % 2)

    fix = pl.pallas_call(
        _fix_kernel,
        out_shape=jax.ShapeDtypeStruct((batch, seq_len), waveform.dtype),
        grid_spec=pltpu.PrefetchScalarGridSpec(
            num_scalar_prefetch=2,
            grid=(batch // 8, 16),
            in_specs=[
                pl.BlockSpec((8, 128), lambda g, t, s_ref, e_ref: (g, 0)),
                pl.BlockSpec((8, 128), lambda g, t, s_ref, e_ref: (g, 0)),
                pl.BlockSpec((8, _BLK), _data_idx),
                pl.BlockSpec(memory_space=pl.ANY),
            ],
            out_specs=pl.BlockSpec((8, _BLK), _data_idx),
        ),
        input_output_aliases={5: 0},
    )
    return fix(s, e, s2d, e2d, waveform, cp)


# speedup vs baseline: 2.1092x; 1.9406x over previous
import jax
import jax.numpy as jnp
from jax import lax
from jax.experimental import pallas as pl
from jax.experimental.pallas import tpu as pltpu

_BATCH = 64
_SEQ_LEN = 262144
_MAX_DROP_LENGTH = 2048
_WIN = _MAX_DROP_LENGTH + 128


def _drop_bounds(batch, seq_len):
    rkey = jax.random.key(42)
    k_start, k_len = jax.random.split(rkey)
    drop_start = jax.random.randint(k_start, (batch,), 0, seq_len // 2)
    drop_len = jax.random.randint(k_len, (batch,), 0, _MAX_DROP_LENGTH)
    drop_end = jnp.minimum(drop_start + drop_len, seq_len)
    return drop_start.astype(jnp.int32), drop_end.astype(jnp.int32)


def _fix_kernel(s_ref, e_ref, ca_ref, x_hbm, cp_any, o_hbm,
                scratch, sem_in, sem_out):
    del cp_any

    def in_copy(r):
        ca = pl.multiple_of(ca_ref[r], 128)
        return pltpu.make_async_copy(
            x_hbm.at[r, pl.ds(ca, _WIN)], scratch.at[r], sem_in.at[r])

    def out_copy(r):
        ca = pl.multiple_of(ca_ref[r], 128)
        return pltpu.make_async_copy(
            scratch.at[r], o_hbm.at[r, pl.ds(ca, _WIN)], sem_out.at[r])

    for r in range(_BATCH):
        in_copy(r).start()
    for r in range(_BATCH):
        in_copy(r).wait()
        s = s_ref[r]
        e = e_ref[r]
        col = ca_ref[r] + lax.broadcasted_iota(jnp.int32, (1, _WIN), 1)
        mask = (col >= s) & (col < e)
        v = scratch[r:r + 1, :]
        scratch[r:r + 1, :] = jnp.where(mask, jnp.zeros((), v.dtype), v)
        out_copy(r).start()
    for r in range(_BATCH):
        out_copy(r).wait()


def kernel(waveform):
    batch, seq_len = waveform.shape
    s, e = _drop_bounds(batch, seq_len)
    ca = (s // 128) * 128

    cp = jax.freeze(jax.new_ref(waveform))

    fix = pl.pallas_call(
        _fix_kernel,
        out_shape=jax.ShapeDtypeStruct((batch, seq_len), waveform.dtype),
        grid_spec=pltpu.PrefetchScalarGridSpec(
            num_scalar_prefetch=3,
            grid=(1,),
            in_specs=[
                pl.BlockSpec(memory_space=pl.ANY),
                pl.BlockSpec(memory_space=pl.ANY),
            ],
            out_specs=pl.BlockSpec(memory_space=pl.ANY),
            scratch_shapes=[
                pltpu.VMEM((_BATCH, _WIN), jnp.float32),
                pltpu.SemaphoreType.DMA((_BATCH,)),
                pltpu.SemaphoreType.DMA((_BATCH,)),
            ],
        ),
        input_output_aliases={4: 0},
    )
    return fix(s, e, ca, waveform, cp)
